# speedup vs baseline: 43.2157x; 43.2157x over previous
import jax
import jax.numpy as jnp
from jax.experimental import pallas as pl


ROWS_PER_BLOCK = 8
MAX_NEWTON_ITERS = 48


def _sparsemax_block(x_ref, o_ref):
    x = x_ref[...]
    m = jnp.max(x, axis=1, keepdims=True)
    tau0 = m - 1.0

    def newton_body(carry):
        i, tau, _ = carry
        mask = x > tau
        k = jnp.sum(mask.astype(jnp.float32), axis=1, keepdims=True)
        s = jnp.sum(jnp.where(mask, x, 0.0), axis=1, keepdims=True)
        k = jnp.maximum(k, 1.0)
        tau_new = (s - 1.0) / k
        changed = jnp.any(tau_new != tau)
        return i + 1, tau_new, changed

    def newton_cond(carry):
        i, _, changed = carry
        return jnp.logical_and(i < MAX_NEWTON_ITERS, changed)

    _, tau, _ = jax.lax.while_loop(
        newton_cond, newton_body, (jnp.int32(0), tau0, jnp.bool_(True))
    )
    o_ref[...] = jnp.maximum(x - tau, 0.0)


def kernel(input):
    rows, n = input.shape
    grid = rows // ROWS_PER_BLOCK
    return pl.pallas_call(
        _sparsemax_block,
        grid=(grid,),
        in_specs=[pl.BlockSpec((ROWS_PER_BLOCK, n), lambda i: (i, 0))],
        out_specs=pl.BlockSpec((ROWS_PER_BLOCK, n), lambda i: (i, 0)),
        out_shape=jax.ShapeDtypeStruct((rows, n), jnp.float32),
    )(input)
